# baseline (device time: 91726 ns/iter reference)
import jax
import jax.numpy as jnp
from jax import lax
from jax.experimental import pallas as pl
from jax.experimental.pallas import tpu as pltpu

B, S, H, Dh, Dr = 1, 1024, 16, 128, 32
D = 2048
DC = 256
DC_SH = 128
SCALE = (Dh + Dr) ** -0.5
F32 = jnp.float32
BF16 = jnp.bfloat16
HPAIR = H // 2


def _proj_comm_body(x_ref, wdkv_ref, wuk_ref, wuv_ref,
                    wq_hbm, wqr_hbm, wkr_hbm,
                    q_ref, qr_ref, kr_ref, c_ref, wukf_ref, wuvf_ref,
                    wq_v, wqr_v, wkr_v, load_sems, send_sems, recv_sems):
    my_x = lax.axis_index("x")
    my_y = lax.axis_index("y")
    peer = (my_x, 1 - my_y)
    off = my_y * DC_SH

    ld_qr = pltpu.make_async_copy(wqr_hbm, wqr_v, load_sems.at[0])
    ld_kr = pltpu.make_async_copy(wkr_hbm, wkr_v, load_sems.at[1])
    ld_q = pltpu.make_async_copy(wq_hbm, wq_v, load_sems.at[2])
    ld_qr.start()
    ld_kr.start()
    ld_q.start()

    wukf_ref[pl.ds(off, DC_SH), :] = wuk_ref[...].astype(BF16)
    wuvf_ref[pl.ds(off, DC_SH), :] = wuv_ref[...].astype(BF16)

    barrier = pltpu.get_barrier_semaphore()
    pl.semaphore_signal(barrier, inc=1, device_id=peer,
                        device_id_type=pl.DeviceIdType.MESH)
    pl.semaphore_wait(barrier, 1)

    rdmas = []
    for i, src in enumerate([wukf_ref.at[pl.ds(off, DC_SH), :],
                             wuvf_ref.at[pl.ds(off, DC_SH), :]]):
        rdma = pltpu.make_async_remote_copy(
            src_ref=src, dst_ref=src,
            send_sem=send_sems.at[i], recv_sem=recv_sems.at[i],
            device_id=peer, device_id_type=pl.DeviceIdType.MESH,
        )
        rdma.start()
        rdmas.append(rdma)

    x = x_ref[...]
    cpart = jnp.dot(x, wdkv_ref[...], preferred_element_type=F32)
    c_ref[:, pl.ds(off, DC_SH)] = cpart.astype(BF16)
    c_src = c_ref.at[:, pl.ds(off, DC_SH)]
    rdma_c = pltpu.make_async_remote_copy(
        src_ref=c_src, dst_ref=c_src,
        send_sem=send_sems.at[2], recv_sem=recv_sems.at[2],
        device_id=peer, device_id_type=pl.DeviceIdType.MESH,
    )
    rdma_c.start()
    rdmas.append(rdma_c)

    ld_qr.wait()
    qr_ref[...] = jnp.dot(x, wqr_v[...], preferred_element_type=F32)
    ld_kr.wait()
    kr_ref[...] = jnp.dot(x, wkr_v[...], preferred_element_type=F32)
    ld_q.wait()
    q_ref[...] = jnp.dot(x, wq_v[...], preferred_element_type=F32)

    for rdma in rdmas:
        rdma.wait()


def _proj_comm(x2d, wdkv_sh, wuk_sh, wuv_sh, wq, wqr, wkr):
    return pl.pallas_call(
        _proj_comm_body,
        out_shape=(
            jax.ShapeDtypeStruct((S, D), F32),
            jax.ShapeDtypeStruct((S, H * Dr), F32),
            jax.ShapeDtypeStruct((S, Dr), F32),
            jax.ShapeDtypeStruct((S, DC), BF16),
            jax.ShapeDtypeStruct((DC, D), BF16),
            jax.ShapeDtypeStruct((DC, D), BF16),
        ),
        in_specs=[pl.BlockSpec(memory_space=pltpu.VMEM)] * 4
        + [pl.BlockSpec(memory_space=pl.ANY)] * 3,
        out_specs=(pl.BlockSpec(memory_space=pltpu.VMEM),) * 6,
        scratch_shapes=[
            pltpu.VMEM((D, D), F32),
            pltpu.VMEM((D, H * Dr), F32),
            pltpu.VMEM((D, Dr), F32),
            pltpu.SemaphoreType.DMA((3,)),
            pltpu.SemaphoreType.DMA((3,)),
            pltpu.SemaphoreType.DMA((3,)),
        ],
        compiler_params=pltpu.CompilerParams(
            collective_id=0, vmem_limit_bytes=62 * 1024 * 1024),
    )(x2d, wdkv_sh, wuk_sh, wuv_sh, wq, wqr, wkr)


TCHUNK = 512
NT = S // TCHUNK


def _attn_body(q_ref, qr_ref, kr_ref, c_ref, wuk_ref, wuv_ref, wo_ref,
               out_ref):
    g = pl.program_id(0)
    c = c_ref[...]
    k2 = jnp.dot(c, wuk_ref[...], preferred_element_type=F32)
    v2 = jnp.dot(c, wuv_ref[...], preferred_element_type=F32)
    kr = kr_ref[...]
    ones = jnp.ones((S, 1), F32)

    o2 = []
    for j in range(2):
        qa = jnp.concatenate(
            [q_ref[:, j * Dh:(j + 1) * Dh], qr_ref[j]], axis=1) * SCALE
        ka = jnp.concatenate(
            [k2[:, j * Dh:(j + 1) * Dh], kr], axis=1)
        vaug = jnp.concatenate(
            [v2[:, j * Dh:(j + 1) * Dh], ones], axis=1)
        oa = jnp.zeros((S, Dh + 1), F32)
        for t in range(NT):
            sl = slice(t * TCHUNK, (t + 1) * TCHUNK)
            s_t = lax.dot_general(qa, ka[sl, :], (((1,), (1,)), ((), ())),
                                  preferred_element_type=F32)
            p_t = jnp.exp(s_t)
            oa = oa + lax.dot_general(p_t, vaug[sl, :],
                                      (((1,), (0,)), ((), ())),
                                      preferred_element_type=F32)
        o2.append(oa[:, :Dh] / oa[:, Dh:Dh + 1])
    opair = jnp.concatenate(o2, axis=1)

    contrib = jnp.dot(opair, wo_ref[...], preferred_element_type=F32)

    @pl.when(g == 0)
    def _():
        out_ref[...] = contrib

    @pl.when(g != 0)
    def _():
        out_ref[...] += contrib


def _attn(q, qr3, kr, c, wukf, wuvf, wo):
    return pl.pallas_call(
        _attn_body,
        grid=(HPAIR,),
        in_specs=[
            pl.BlockSpec((S, 2 * Dh), lambda g: (0, g)),
            pl.BlockSpec((2, S, Dr), lambda g: (g, 0, 0)),
            pl.BlockSpec((S, Dr), lambda g: (0, 0)),
            pl.BlockSpec((S, DC), lambda g: (0, 0)),
            pl.BlockSpec((DC, 2 * Dh), lambda g: (0, g)),
            pl.BlockSpec((DC, 2 * Dh), lambda g: (0, g)),
            pl.BlockSpec((2 * Dh, D), lambda g: (g, 0)),
        ],
        out_specs=pl.BlockSpec((S, D), lambda g: (0, 0)),
        out_shape=jax.ShapeDtypeStruct((S, D), F32),
        compiler_params=pltpu.CompilerParams(
            vmem_limit_bytes=62 * 1024 * 1024),
    )(q, qr3, kr, c, wukf, wuvf, wo)


def kernel(x, Wdkv, Wuk, Wuv, Wq, Wqr, Wkr, Wo):
    x2d = x.reshape(S, D)
    q, qr, kr, c, wukf, wuvf = _proj_comm(x2d, Wdkv, Wuk, Wuv, Wq, Wqr, Wkr)
    qr3 = qr.reshape(S, H, Dr).transpose(1, 0, 2)
    out = _attn(q, qr3, kr, c, wukf, wuvf, Wo)
    return out.reshape(B, S, D)


# device time: 85055 ns/iter; 1.0784x vs baseline; 1.0784x over previous
import jax
import jax.numpy as jnp
from jax import lax
from jax.experimental import pallas as pl
from jax.experimental.pallas import tpu as pltpu

B, S, H, Dh, Dr = 1, 1024, 16, 128, 32
D = 2048
DC = 256
DC_SH = 128
SCALE = (Dh + Dr) ** -0.5
F32 = jnp.float32
BF16 = jnp.bfloat16
HPAIR = H // 2


def _proj_comm_body(x_ref, wdkv_ref, wuk_ref, wuv_ref,
                    wq_hbm, wqr_hbm, wkr_hbm,
                    q_ref, qr_ref, kr_ref, c_ref, wukf_ref, wuvf_ref,
                    wq_v, wqr_v, wkr_v, load_sems, send_sems, recv_sems):
    my_x = lax.axis_index("x")
    my_y = lax.axis_index("y")
    peer = (my_x, 1 - my_y)
    off = my_y * DC_SH

    ld_qr = pltpu.make_async_copy(wqr_hbm, wqr_v, load_sems.at[0])
    ld_kr = pltpu.make_async_copy(wkr_hbm, wkr_v, load_sems.at[1])
    ld_q = pltpu.make_async_copy(wq_hbm, wq_v, load_sems.at[2])
    ld_qr.start()
    ld_kr.start()
    ld_q.start()

    wukf_ref[pl.ds(off, DC_SH), :] = wuk_ref[...].astype(BF16)
    wuvf_ref[pl.ds(off, DC_SH), :] = wuv_ref[...].astype(BF16)

    barrier = pltpu.get_barrier_semaphore()
    pl.semaphore_signal(barrier, inc=1, device_id=peer,
                        device_id_type=pl.DeviceIdType.MESH)
    pl.semaphore_wait(barrier, 1)

    rdmas = []
    for i, src in enumerate([wukf_ref.at[pl.ds(off, DC_SH), :],
                             wuvf_ref.at[pl.ds(off, DC_SH), :]]):
        rdma = pltpu.make_async_remote_copy(
            src_ref=src, dst_ref=src,
            send_sem=send_sems.at[i], recv_sem=recv_sems.at[i],
            device_id=peer, device_id_type=pl.DeviceIdType.MESH,
        )
        rdma.start()
        rdmas.append(rdma)

    x = x_ref[...]
    cpart = jnp.dot(x, wdkv_ref[...], preferred_element_type=F32)
    c_ref[:, pl.ds(off, DC_SH)] = cpart.astype(BF16)
    c_src = c_ref.at[:, pl.ds(off, DC_SH)]
    rdma_c = pltpu.make_async_remote_copy(
        src_ref=c_src, dst_ref=c_src,
        send_sem=send_sems.at[2], recv_sem=recv_sems.at[2],
        device_id=peer, device_id_type=pl.DeviceIdType.MESH,
    )
    rdma_c.start()
    rdmas.append(rdma_c)

    ld_qr.wait()
    qr_ref[...] = jnp.dot(x, wqr_v[...], preferred_element_type=F32)
    ld_kr.wait()
    kr_ref[...] = jnp.dot(x, wkr_v[...], preferred_element_type=F32)
    ld_q.wait()
    q_ref[...] = jnp.dot(x, wq_v[...], preferred_element_type=F32)

    for rdma in rdmas:
        rdma.wait()


def _proj_comm(x2d, wdkv_sh, wuk_sh, wuv_sh, wq, wqr, wkr):
    return pl.pallas_call(
        _proj_comm_body,
        out_shape=(
            jax.ShapeDtypeStruct((S, D), F32),
            jax.ShapeDtypeStruct((S, H * Dr), F32),
            jax.ShapeDtypeStruct((S, Dr), F32),
            jax.ShapeDtypeStruct((S, DC), BF16),
            jax.ShapeDtypeStruct((DC, D), BF16),
            jax.ShapeDtypeStruct((DC, D), BF16),
        ),
        in_specs=[pl.BlockSpec(memory_space=pltpu.VMEM)] * 4
        + [pl.BlockSpec(memory_space=pl.ANY)] * 3,
        out_specs=(pl.BlockSpec(memory_space=pltpu.VMEM),) * 6,
        scratch_shapes=[
            pltpu.VMEM((D, D), F32),
            pltpu.VMEM((D, H * Dr), F32),
            pltpu.VMEM((D, Dr), F32),
            pltpu.SemaphoreType.DMA((3,)),
            pltpu.SemaphoreType.DMA((3,)),
            pltpu.SemaphoreType.DMA((3,)),
        ],
        compiler_params=pltpu.CompilerParams(
            collective_id=0, vmem_limit_bytes=62 * 1024 * 1024),
    )(x2d, wdkv_sh, wuk_sh, wuv_sh, wq, wqr, wkr)


TCHUNK = 512
NT = S // TCHUNK


def _attn_body(q_ref, qr_ref, kr_ref, c_ref, wuk_ref, wuv_ref, wo_hbm,
               out_ref, oacc_ref, wo_v, wo_sem):
    g = pl.program_id(0)

    @pl.when(g == 0)
    def _():
        pltpu.make_async_copy(wo_hbm, wo_v, wo_sem).start()

    c = c_ref[...]
    k2 = jnp.dot(c, wuk_ref[...], preferred_element_type=F32)
    v2 = jnp.dot(c, wuv_ref[...], preferred_element_type=F32)
    kr = kr_ref[...]
    ones = jnp.ones((S, 1), F32)

    o2 = []
    for j in range(2):
        qa = jnp.concatenate(
            [q_ref[:, j * Dh:(j + 1) * Dh], qr_ref[j]], axis=1) * SCALE
        ka = jnp.concatenate(
            [k2[:, j * Dh:(j + 1) * Dh], kr], axis=1)
        vaug = jnp.concatenate(
            [v2[:, j * Dh:(j + 1) * Dh], ones], axis=1)
        oa = jnp.zeros((S, Dh + 1), F32)
        for t in range(NT):
            sl = slice(t * TCHUNK, (t + 1) * TCHUNK)
            s_t = lax.dot_general(qa, ka[sl, :], (((1,), (1,)), ((), ())),
                                  preferred_element_type=F32)
            p_t = jnp.exp(s_t)
            oa = oa + lax.dot_general(p_t, vaug[sl, :],
                                      (((1,), (0,)), ((), ())),
                                      preferred_element_type=F32)
        o2.append(oa[:, :Dh] / oa[:, Dh:Dh + 1])
    oacc_ref[:, pl.ds(g * 2 * Dh, 2 * Dh)] = jnp.concatenate(o2, axis=1)

    @pl.when(g == HPAIR - 1)
    def _():
        pltpu.make_async_copy(wo_hbm, wo_v, wo_sem).wait()
        out_ref[...] = jnp.dot(oacc_ref[...], wo_v[...],
                               preferred_element_type=F32)


def _attn(q, qr3, kr, c, wukf, wuvf, wo):
    return pl.pallas_call(
        _attn_body,
        grid=(HPAIR,),
        in_specs=[
            pl.BlockSpec((S, 2 * Dh), lambda g: (0, g)),
            pl.BlockSpec((2, S, Dr), lambda g: (g, 0, 0)),
            pl.BlockSpec((S, Dr), lambda g: (0, 0)),
            pl.BlockSpec((S, DC), lambda g: (0, 0)),
            pl.BlockSpec((DC, 2 * Dh), lambda g: (0, g)),
            pl.BlockSpec((DC, 2 * Dh), lambda g: (0, g)),
            pl.BlockSpec(memory_space=pl.ANY),
        ],
        out_specs=pl.BlockSpec((S, D), lambda g: (0, 0)),
        out_shape=jax.ShapeDtypeStruct((S, D), F32),
        scratch_shapes=[
            pltpu.VMEM((S, D), F32),
            pltpu.VMEM((D, D), F32),
            pltpu.SemaphoreType.DMA,
        ],
        compiler_params=pltpu.CompilerParams(
            vmem_limit_bytes=62 * 1024 * 1024),
    )(q, qr3, kr, c, wukf, wuvf, wo)


def kernel(x, Wdkv, Wuk, Wuv, Wq, Wqr, Wkr, Wo):
    x2d = x.reshape(S, D)
    q, qr, kr, c, wukf, wuvf = _proj_comm(x2d, Wdkv, Wuk, Wuv, Wq, Wqr, Wkr)
    qr3 = qr.reshape(S, H, Dr).transpose(1, 0, 2)
    out = _attn(q, qr3, kr, c, wukf, wuvf, Wo)
    return out.reshape(B, S, D)


# device time: 81460 ns/iter; 1.1260x vs baseline; 1.0441x over previous
import jax
import jax.numpy as jnp
from jax import lax
from jax.experimental import pallas as pl
from jax.experimental.pallas import tpu as pltpu

B, S, H, Dh, Dr = 1, 1024, 16, 128, 32
D = 2048
DC = 256
DC_SH = 128
SCALE = (Dh + Dr) ** -0.5
F32 = jnp.float32
BF16 = jnp.bfloat16
HPAIR = H // 2


def _proj_comm_body(x_ref, wdkv_ref, wuk_ref, wuv_ref,
                    wq_hbm, wqr_hbm, wkr_hbm,
                    q_ref, qr_ref, kr_ref, c_ref, wukf_ref, wuvf_ref,
                    wq_v, wqr_v, wkr_v, load_sems, send_sems, recv_sems):
    my_x = lax.axis_index("x")
    my_y = lax.axis_index("y")
    peer = (my_x, 1 - my_y)
    off = my_y * DC_SH

    ld_qr = pltpu.make_async_copy(wqr_hbm, wqr_v, load_sems.at[0])
    ld_kr = pltpu.make_async_copy(wkr_hbm, wkr_v, load_sems.at[1])
    ld_q = pltpu.make_async_copy(wq_hbm, wq_v, load_sems.at[2])
    ld_qr.start()
    ld_kr.start()
    ld_q.start()

    wukf_ref[pl.ds(off, DC_SH), :] = wuk_ref[...].astype(BF16)
    wuvf_ref[pl.ds(off, DC_SH), :] = wuv_ref[...].astype(BF16)

    barrier = pltpu.get_barrier_semaphore()
    pl.semaphore_signal(barrier, inc=1, device_id=peer,
                        device_id_type=pl.DeviceIdType.MESH)
    pl.semaphore_wait(barrier, 1)

    rdmas = []
    for i, src in enumerate([wukf_ref.at[pl.ds(off, DC_SH), :],
                             wuvf_ref.at[pl.ds(off, DC_SH), :]]):
        rdma = pltpu.make_async_remote_copy(
            src_ref=src, dst_ref=src,
            send_sem=send_sems.at[i], recv_sem=recv_sems.at[i],
            device_id=peer, device_id_type=pl.DeviceIdType.MESH,
        )
        rdma.start()
        rdmas.append(rdma)

    x = x_ref[...]
    cpart = jnp.dot(x, wdkv_ref[...], preferred_element_type=F32)
    c_ref[:, pl.ds(off, DC_SH)] = cpart.astype(BF16)
    c_src = c_ref.at[:, pl.ds(off, DC_SH)]
    rdma_c = pltpu.make_async_remote_copy(
        src_ref=c_src, dst_ref=c_src,
        send_sem=send_sems.at[2], recv_sem=recv_sems.at[2],
        device_id=peer, device_id_type=pl.DeviceIdType.MESH,
    )
    rdma_c.start()
    rdmas.append(rdma_c)

    ld_qr.wait()
    qr_ref[...] = jnp.dot(x, wqr_v[...], preferred_element_type=F32)
    ld_kr.wait()
    kr_ref[...] = jnp.dot(x, wkr_v[...], preferred_element_type=F32)
    ld_q.wait()
    q_ref[...] = jnp.dot(x, wq_v[...], preferred_element_type=F32)

    for rdma in rdmas:
        rdma.wait()


def _proj_comm(x2d, wdkv_sh, wuk_sh, wuv_sh, wq, wqr, wkr):
    return pl.pallas_call(
        _proj_comm_body,
        out_shape=(
            jax.ShapeDtypeStruct((S, D), F32),
            jax.ShapeDtypeStruct((S, H * Dr), F32),
            jax.ShapeDtypeStruct((S, Dr), F32),
            jax.ShapeDtypeStruct((S, DC), BF16),
            jax.ShapeDtypeStruct((DC, D), BF16),
            jax.ShapeDtypeStruct((DC, D), BF16),
        ),
        in_specs=[pl.BlockSpec(memory_space=pltpu.VMEM)] * 4
        + [pl.BlockSpec(memory_space=pl.ANY)] * 3,
        out_specs=(pl.BlockSpec(memory_space=pltpu.VMEM),) * 6,
        scratch_shapes=[
            pltpu.VMEM((D, D), F32),
            pltpu.VMEM((D, H * Dr), F32),
            pltpu.VMEM((D, Dr), F32),
            pltpu.SemaphoreType.DMA((3,)),
            pltpu.SemaphoreType.DMA((3,)),
            pltpu.SemaphoreType.DMA((3,)),
        ],
        compiler_params=pltpu.CompilerParams(
            collective_id=0, vmem_limit_bytes=62 * 1024 * 1024),
    )(x2d, wdkv_sh, wuk_sh, wuv_sh, wq, wqr, wkr)


TCHUNK = 512
NT = S // TCHUNK


def _attn_body(q_ref, qr_ref, kr_ref, c_ref, wuk_ref, wuv_ref, wo_hbm,
               out_ref, oacc_ref, wo_v, wo_sem):
    g = pl.program_id(0)

    @pl.when(g == 0)
    def _():
        pltpu.make_async_copy(wo_hbm, wo_v, wo_sem).start()

    c = c_ref[...]
    k2 = jnp.dot(c, wuk_ref[...], preferred_element_type=F32)
    v2 = jnp.dot(c, wuv_ref[...], preferred_element_type=F32)
    kr = kr_ref[...]
    ones = jnp.ones((S, 1), F32)

    o2 = []
    for j in range(2):
        qa = jnp.concatenate(
            [q_ref[:, j * Dh:(j + 1) * Dh], qr_ref[j]], axis=1) * SCALE
        ka = jnp.concatenate(
            [k2[:, j * Dh:(j + 1) * Dh], kr], axis=1)
        vaug = jnp.concatenate(
            [v2[:, j * Dh:(j + 1) * Dh], ones], axis=1)
        s = lax.dot_general(qa, ka, (((1,), (1,)), ((), ())),
                            preferred_element_type=F32)
        p = jnp.exp(s)
        oa = lax.dot_general(p, vaug, (((1,), (0,)), ((), ())),
                             preferred_element_type=F32)
        o2.append(oa[:, :Dh] / oa[:, Dh:Dh + 1])
    oacc_ref[:, pl.ds(g * 2 * Dh, 2 * Dh)] = jnp.concatenate(o2, axis=1)

    @pl.when(g == HPAIR - 1)
    def _():
        pltpu.make_async_copy(wo_hbm, wo_v, wo_sem).wait()
        out_ref[...] = jnp.dot(oacc_ref[...], wo_v[...],
                               preferred_element_type=F32)


def _attn(q, qr3, kr, c, wukf, wuvf, wo):
    return pl.pallas_call(
        _attn_body,
        grid=(HPAIR,),
        in_specs=[
            pl.BlockSpec((S, 2 * Dh), lambda g: (0, g)),
            pl.BlockSpec((2, S, Dr), lambda g: (g, 0, 0)),
            pl.BlockSpec((S, Dr), lambda g: (0, 0)),
            pl.BlockSpec((S, DC), lambda g: (0, 0)),
            pl.BlockSpec((DC, 2 * Dh), lambda g: (0, g)),
            pl.BlockSpec((DC, 2 * Dh), lambda g: (0, g)),
            pl.BlockSpec(memory_space=pl.ANY),
        ],
        out_specs=pl.BlockSpec((S, D), lambda g: (0, 0)),
        out_shape=jax.ShapeDtypeStruct((S, D), F32),
        scratch_shapes=[
            pltpu.VMEM((S, D), F32),
            pltpu.VMEM((D, D), F32),
            pltpu.SemaphoreType.DMA,
        ],
        compiler_params=pltpu.CompilerParams(
            vmem_limit_bytes=62 * 1024 * 1024),
    )(q, qr3, kr, c, wukf, wuvf, wo)


def kernel(x, Wdkv, Wuk, Wuv, Wq, Wqr, Wkr, Wo):
    x2d = x.reshape(S, D)
    q, qr, kr, c, wukf, wuvf = _proj_comm(x2d, Wdkv, Wuk, Wuv, Wq, Wqr, Wkr)
    qr3 = qr.reshape(S, H, Dr).transpose(1, 0, 2)
    out = _attn(q, qr3, kr, c, wukf, wuvf, Wo)
    return out.reshape(B, S, D)


# device time: 78320 ns/iter; 1.1712x vs baseline; 1.0401x over previous
import jax
import jax.numpy as jnp
from jax import lax
from jax.experimental import pallas as pl
from jax.experimental.pallas import tpu as pltpu

B, S, H, Dh, Dr = 1, 1024, 16, 128, 32
D = 2048
DC = 256
DC_SH = 128
SCALE = (Dh + Dr) ** -0.5
F32 = jnp.float32
BF16 = jnp.bfloat16
HPAIR = H // 2


def _proj_comm_body(x_ref, wdkv_ref, wuk_ref, wuv_ref,
                    wq_hbm, wqr_hbm, wkr_hbm,
                    q_ref, qr_ref, kr_ref, c_ref, wukf_ref, wuvf_ref,
                    wq_v, wqr_v, wkr_v, load_sems, send_sems, recv_sems):
    my_x = lax.axis_index("x")
    my_y = lax.axis_index("y")
    peer = (my_x, 1 - my_y)
    off = my_y * DC_SH

    ld_qr = pltpu.make_async_copy(wqr_hbm, wqr_v, load_sems.at[0])
    ld_kr = pltpu.make_async_copy(wkr_hbm, wkr_v, load_sems.at[1])
    ld_q = pltpu.make_async_copy(wq_hbm, wq_v, load_sems.at[2])
    ld_qr.start()
    ld_kr.start()
    ld_q.start()

    wukf_ref[pl.ds(off, DC_SH), :] = wuk_ref[...].astype(BF16)
    wuvf_ref[pl.ds(off, DC_SH), :] = wuv_ref[...].astype(BF16)

    barrier = pltpu.get_barrier_semaphore()
    pl.semaphore_signal(barrier, inc=1, device_id=peer,
                        device_id_type=pl.DeviceIdType.MESH)
    pl.semaphore_wait(barrier, 1)

    rdmas = []
    for i, src in enumerate([wukf_ref.at[pl.ds(off, DC_SH), :],
                             wuvf_ref.at[pl.ds(off, DC_SH), :]]):
        rdma = pltpu.make_async_remote_copy(
            src_ref=src, dst_ref=src,
            send_sem=send_sems.at[i], recv_sem=recv_sems.at[i],
            device_id=peer, device_id_type=pl.DeviceIdType.MESH,
        )
        rdma.start()
        rdmas.append(rdma)

    x = x_ref[...]
    cpart = jnp.dot(x, wdkv_ref[...], preferred_element_type=F32)
    c_ref[:, pl.ds(off, DC_SH)] = cpart.astype(BF16)
    c_src = c_ref.at[:, pl.ds(off, DC_SH)]
    rdma_c = pltpu.make_async_remote_copy(
        src_ref=c_src, dst_ref=c_src,
        send_sem=send_sems.at[2], recv_sem=recv_sems.at[2],
        device_id=peer, device_id_type=pl.DeviceIdType.MESH,
    )
    rdma_c.start()
    rdmas.append(rdma_c)

    ld_qr.wait()
    qr2d = jnp.dot(x, wqr_v[...], preferred_element_type=F32)
    for h in range(H):
        qr_ref[h] = qr2d[:, h * Dr:(h + 1) * Dr]
    ld_kr.wait()
    kr_ref[...] = jnp.dot(x, wkr_v[...], preferred_element_type=F32)
    ld_q.wait()
    q_ref[...] = jnp.dot(x, wq_v[...], preferred_element_type=F32)

    for rdma in rdmas:
        rdma.wait()


def _proj_comm(x2d, wdkv_sh, wuk_sh, wuv_sh, wq, wqr, wkr):
    return pl.pallas_call(
        _proj_comm_body,
        out_shape=(
            jax.ShapeDtypeStruct((S, D), F32),
            jax.ShapeDtypeStruct((H, S, Dr), F32),
            jax.ShapeDtypeStruct((S, Dr), F32),
            jax.ShapeDtypeStruct((S, DC), BF16),
            jax.ShapeDtypeStruct((DC, D), BF16),
            jax.ShapeDtypeStruct((DC, D), BF16),
        ),
        in_specs=[pl.BlockSpec(memory_space=pltpu.VMEM)] * 4
        + [pl.BlockSpec(memory_space=pl.ANY)] * 3,
        out_specs=(pl.BlockSpec(memory_space=pltpu.VMEM),) * 6,
        scratch_shapes=[
            pltpu.VMEM((D, D), F32),
            pltpu.VMEM((D, H * Dr), F32),
            pltpu.VMEM((D, Dr), F32),
            pltpu.SemaphoreType.DMA((3,)),
            pltpu.SemaphoreType.DMA((3,)),
            pltpu.SemaphoreType.DMA((3,)),
        ],
        compiler_params=pltpu.CompilerParams(
            collective_id=0, vmem_limit_bytes=62 * 1024 * 1024),
    )(x2d, wdkv_sh, wuk_sh, wuv_sh, wq, wqr, wkr)


TCHUNK = 512
NT = S // TCHUNK


def _attn_body(q_ref, qr_ref, kr_ref, c_ref, wuk_ref, wuv_ref, wo_hbm,
               out_ref, oacc_ref, wo_v, wo_sem):
    g = pl.program_id(0)

    @pl.when(g == 0)
    def _():
        pltpu.make_async_copy(wo_hbm, wo_v, wo_sem).start()

    c = c_ref[...]
    k2 = jnp.dot(c, wuk_ref[...], preferred_element_type=F32)
    v2 = jnp.dot(c, wuv_ref[...], preferred_element_type=F32)
    kr = kr_ref[...]
    ones = jnp.ones((S, 1), F32)

    o2 = []
    for j in range(2):
        qa = jnp.concatenate(
            [q_ref[:, j * Dh:(j + 1) * Dh], qr_ref[j]],
            axis=1) * (SCALE * 1.4426950408889634)
        ka = jnp.concatenate(
            [k2[:, j * Dh:(j + 1) * Dh], kr], axis=1)
        vaug = jnp.concatenate(
            [v2[:, j * Dh:(j + 1) * Dh], ones], axis=1)
        s = lax.dot_general(qa, ka, (((1,), (1,)), ((), ())),
                            preferred_element_type=F32)
        p = jnp.exp2(s)
        oa = lax.dot_general(p, vaug, (((1,), (0,)), ((), ())),
                             preferred_element_type=F32)
        o2.append(oa[:, :Dh] / oa[:, Dh:Dh + 1])
    oacc_ref[:, pl.ds(g * 2 * Dh, 2 * Dh)] = jnp.concatenate(o2, axis=1)

    @pl.when(g == HPAIR - 1)
    def _():
        pltpu.make_async_copy(wo_hbm, wo_v, wo_sem).wait()
        out_ref[...] = jnp.dot(oacc_ref[...], wo_v[...],
                               preferred_element_type=F32)


def _attn(q, qr3, kr, c, wukf, wuvf, wo):
    return pl.pallas_call(
        _attn_body,
        grid=(HPAIR,),
        in_specs=[
            pl.BlockSpec((S, 2 * Dh), lambda g: (0, g)),
            pl.BlockSpec((2, S, Dr), lambda g: (g, 0, 0)),
            pl.BlockSpec((S, Dr), lambda g: (0, 0)),
            pl.BlockSpec((S, DC), lambda g: (0, 0)),
            pl.BlockSpec((DC, 2 * Dh), lambda g: (0, g)),
            pl.BlockSpec((DC, 2 * Dh), lambda g: (0, g)),
            pl.BlockSpec(memory_space=pl.ANY),
        ],
        out_specs=pl.BlockSpec((S, D), lambda g: (0, 0)),
        out_shape=jax.ShapeDtypeStruct((S, D), F32),
        scratch_shapes=[
            pltpu.VMEM((S, D), F32),
            pltpu.VMEM((D, D), F32),
            pltpu.SemaphoreType.DMA,
        ],
        compiler_params=pltpu.CompilerParams(
            vmem_limit_bytes=62 * 1024 * 1024),
    )(q, qr3, kr, c, wukf, wuvf, wo)


def kernel(x, Wdkv, Wuk, Wuv, Wq, Wqr, Wkr, Wo):
    x2d = x.reshape(S, D)
    q, qr3, kr, c, wukf, wuvf = _proj_comm(x2d, Wdkv, Wuk, Wuv, Wq, Wqr, Wkr)
    out = _attn(q, qr3, kr, c, wukf, wuvf, Wo)
    return out.reshape(B, S, D)


# device time: 74438 ns/iter; 1.2322x vs baseline; 1.0522x over previous
import jax
import jax.numpy as jnp
from jax import lax
from jax.experimental import pallas as pl
from jax.experimental.pallas import tpu as pltpu

B, S, H, Dh, Dr = 1, 1024, 16, 128, 32
D = 2048
DC = 256
DC_SH = 128
SCALE = (Dh + Dr) ** -0.5
F32 = jnp.float32
BF16 = jnp.bfloat16
HPAIR = H // 2


def _proj_comm_body(x_ref, wdkv_ref, wuk_ref, wuv_ref,
                    wq_hbm, wqr_hbm, wkr_hbm,
                    q_ref, qr_ref, kr_ref, c_ref, wukf_ref, wuvf_ref,
                    wq_v, wqr_v, wkr_v, load_sems, send_sems, recv_sems):
    my_x = lax.axis_index("x")
    my_y = lax.axis_index("y")
    peer = (my_x, 1 - my_y)
    off = my_y * DC_SH

    ld_qr = pltpu.make_async_copy(wqr_hbm, wqr_v, load_sems.at[0])
    ld_kr = pltpu.make_async_copy(wkr_hbm, wkr_v, load_sems.at[1])
    ld_q = pltpu.make_async_copy(wq_hbm, wq_v, load_sems.at[2])
    ld_qr.start()
    ld_kr.start()
    ld_q.start()

    wukf_ref[pl.ds(off, DC_SH), :] = wuk_ref[...].astype(BF16)
    wuvf_ref[pl.ds(off, DC_SH), :] = wuv_ref[...].astype(BF16)

    barrier = pltpu.get_barrier_semaphore()
    pl.semaphore_signal(barrier, inc=1, device_id=peer,
                        device_id_type=pl.DeviceIdType.MESH)
    pl.semaphore_wait(barrier, 1)

    rdmas = []
    for i, src in enumerate([wukf_ref.at[pl.ds(off, DC_SH), :],
                             wuvf_ref.at[pl.ds(off, DC_SH), :]]):
        rdma = pltpu.make_async_remote_copy(
            src_ref=src, dst_ref=src,
            send_sem=send_sems.at[i], recv_sem=recv_sems.at[i],
            device_id=peer, device_id_type=pl.DeviceIdType.MESH,
        )
        rdma.start()
        rdmas.append(rdma)

    x = x_ref[...]
    cpart = jnp.dot(x, wdkv_ref[...], preferred_element_type=F32)
    c_ref[:, pl.ds(off, DC_SH)] = cpart.astype(BF16)
    c_src = c_ref.at[:, pl.ds(off, DC_SH)]
    rdma_c = pltpu.make_async_remote_copy(
        src_ref=c_src, dst_ref=c_src,
        send_sem=send_sems.at[2], recv_sem=recv_sems.at[2],
        device_id=peer, device_id_type=pl.DeviceIdType.MESH,
    )
    rdma_c.start()
    rdmas.append(rdma_c)

    ld_qr.wait()
    qr2d = jnp.dot(x, wqr_v[...], preferred_element_type=F32)
    for h in range(H):
        qr_ref[h] = qr2d[:, h * Dr:(h + 1) * Dr].astype(BF16)
    ld_kr.wait()
    kr_ref[...] = jnp.dot(x, wkr_v[...], preferred_element_type=F32)
    ld_q.wait()
    q_ref[...] = jnp.dot(x, wq_v[...],
                         preferred_element_type=F32).astype(BF16)

    for rdma in rdmas:
        rdma.wait()


def _proj_comm(x2d, wdkv_sh, wuk_sh, wuv_sh, wq, wqr, wkr):
    return pl.pallas_call(
        _proj_comm_body,
        out_shape=(
            jax.ShapeDtypeStruct((S, D), BF16),
            jax.ShapeDtypeStruct((H, S, Dr), BF16),
            jax.ShapeDtypeStruct((S, Dr), F32),
            jax.ShapeDtypeStruct((S, DC), BF16),
            jax.ShapeDtypeStruct((DC, D), BF16),
            jax.ShapeDtypeStruct((DC, D), BF16),
        ),
        in_specs=[pl.BlockSpec(memory_space=pltpu.VMEM)] * 4
        + [pl.BlockSpec(memory_space=pl.ANY)] * 3,
        out_specs=(pl.BlockSpec(memory_space=pltpu.VMEM),) * 6,
        scratch_shapes=[
            pltpu.VMEM((D, D), F32),
            pltpu.VMEM((D, H * Dr), F32),
            pltpu.VMEM((D, Dr), F32),
            pltpu.SemaphoreType.DMA((3,)),
            pltpu.SemaphoreType.DMA((3,)),
            pltpu.SemaphoreType.DMA((3,)),
        ],
        compiler_params=pltpu.CompilerParams(
            collective_id=0, vmem_limit_bytes=62 * 1024 * 1024),
    )(x2d, wdkv_sh, wuk_sh, wuv_sh, wq, wqr, wkr)


TCHUNK = 512
NT = S // TCHUNK


def _attn_body(q_ref, qr_ref, kr_ref, c_ref, wuk_ref, wuv_ref, wo_hbm,
               out_ref, oacc_ref, wo_v, wo_sem):
    g = pl.program_id(0)

    @pl.when(g == 0)
    def _():
        pltpu.make_async_copy(wo_hbm, wo_v, wo_sem).start()

    c = c_ref[...]
    k2 = jnp.dot(c, wuk_ref[...], preferred_element_type=F32)
    v2 = jnp.dot(c, wuv_ref[...], preferred_element_type=F32)
    kr = kr_ref[...]
    ones = jnp.ones((S, 1), F32)

    o2 = []
    for j in range(2):
        qa = jnp.concatenate(
            [q_ref[:, j * Dh:(j + 1) * Dh], qr_ref[j]],
            axis=1).astype(F32) * (SCALE * 1.4426950408889634)
        ka = jnp.concatenate(
            [k2[:, j * Dh:(j + 1) * Dh], kr], axis=1)
        vaug = jnp.concatenate(
            [v2[:, j * Dh:(j + 1) * Dh], ones], axis=1)
        s = lax.dot_general(qa, ka, (((1,), (1,)), ((), ())),
                            preferred_element_type=F32)
        p = jnp.exp2(s)
        oa = lax.dot_general(p, vaug, (((1,), (0,)), ((), ())),
                             preferred_element_type=F32)
        o2.append(oa[:, :Dh] / oa[:, Dh:Dh + 1])
    oacc_ref[:, pl.ds(g * 2 * Dh, 2 * Dh)] = jnp.concatenate(o2, axis=1)

    @pl.when(g == HPAIR - 1)
    def _():
        pltpu.make_async_copy(wo_hbm, wo_v, wo_sem).wait()
        out_ref[...] = jnp.dot(oacc_ref[...], wo_v[...],
                               preferred_element_type=F32)


def _attn(q, qr3, kr, c, wukf, wuvf, wo):
    return pl.pallas_call(
        _attn_body,
        grid=(HPAIR,),
        in_specs=[
            pl.BlockSpec((S, 2 * Dh), lambda g: (0, g)),
            pl.BlockSpec((2, S, Dr), lambda g: (g, 0, 0)),
            pl.BlockSpec((S, Dr), lambda g: (0, 0)),
            pl.BlockSpec((S, DC), lambda g: (0, 0)),
            pl.BlockSpec((DC, 2 * Dh), lambda g: (0, g)),
            pl.BlockSpec((DC, 2 * Dh), lambda g: (0, g)),
            pl.BlockSpec(memory_space=pl.ANY),
        ],
        out_specs=pl.BlockSpec((S, D), lambda g: (0, 0)),
        out_shape=jax.ShapeDtypeStruct((S, D), F32),
        scratch_shapes=[
            pltpu.VMEM((S, D), F32),
            pltpu.VMEM((D, D), F32),
            pltpu.SemaphoreType.DMA,
        ],
        compiler_params=pltpu.CompilerParams(
            vmem_limit_bytes=62 * 1024 * 1024),
    )(q, qr3, kr, c, wukf, wuvf, wo)


def kernel(x, Wdkv, Wuk, Wuv, Wq, Wqr, Wkr, Wo):
    x2d = x.reshape(S, D)
    q, qr3, kr, c, wukf, wuvf = _proj_comm(x2d, Wdkv, Wuk, Wuv, Wq, Wqr, Wkr)
    out = _attn(q, qr3, kr, c, wukf, wuvf, Wo)
    return out.reshape(B, S, D)
